# Initial kernel scaffold; baseline (speedup 1.0000x reference)
import functools

import jax
import jax.numpy as jnp
from jax import lax
from jax.experimental import pallas as pl
from jax.experimental.pallas import tpu as pltpu
from jax.experimental.pallas import tpu_sc as plsc

GAMMA_ = 0.01
B_, G_, S_, L_ = 64, 4096, 32, 4
LANES = 16
NSLAB = B_ // LANES
NPART = 32 // NSLAB
GPT = G_ // NPART
GC = 64
NCHUNK = GPT // GC

_LN2 = 0.6931471805599453
_INV_GAMMA = 1.0 / GAMMA_


def _sc_log(v):
    bits = plsc.bitcast(v, jnp.int32)
    e = ((bits >> 23) & 0xFF) - 127
    m = plsc.bitcast((bits & 0x007FFFFF) | 0x3F800000, jnp.float32)
    z = (m - 1.0) / (m + 1.0)
    z2 = z * z
    p = 1.0 / 9.0 + z2 * (1.0 / 11.0)
    p = 1.0 / 7.0 + z2 * p
    p = 1.0 / 5.0 + z2 * p
    p = 1.0 / 3.0 + z2 * p
    p = 1.0 + z2 * p
    return e.astype(jnp.float32) * _LN2 + 2.0 * z * p


def _clause_body(xt_hbm, idx_hbm, out_hbm, table_v, idx_v, out_v):
    wid = lax.axis_index("s") * 2 + lax.axis_index("c")
    bslab = lax.rem(wid, NSLAB)
    gpart = lax.div(wid, NSLAB)
    g0 = gpart * GPT

    pltpu.sync_copy(xt_hbm.at[bslab], table_v)

    @pl.loop(0, NCHUNK)
    def _chunk(c):
        gs = g0 + c * GC
        pltpu.sync_copy(idx_hbm.at[pl.ds(gs, GC)], idx_v)

        @pl.loop(0, GC)
        def _atom(gl):
            conj = []
            mx = None
            for s in range(S_):
                c0 = table_v[idx_v[gl, 4 * s + 0]]
                c1 = table_v[idx_v[gl, 4 * s + 1]]
                c2 = table_v[idx_v[gl, 4 * s + 2]]
                c3 = table_v[idx_v[gl, 4 * s + 3]]
                cv = (c0 * c1) * (c2 * c3)
                conj.append(cv)
                mx = cv if mx is None else jnp.maximum(mx, cv)
            acc = None
            for s in range(S_):
                ev = jnp.exp((conj[s] - mx) * _INV_GAMMA)
                acc = ev if acc is None else acc + ev
            out_v[gl, :] = mx + GAMMA_ * _sc_log(acc)

        pltpu.sync_copy(out_v, out_hbm.at[pl.ds(wid * GPT + c * GC, GC)])


@jax.jit
def kernel(x, I_i):
    xt = x.T.reshape(G_, NSLAB, LANES).transpose(1, 0, 2)
    idx = I_i.astype(jnp.int32).reshape(G_, S_ * L_)

    mesh = plsc.VectorSubcoreMesh(core_axis_name="c", subcore_axis_name="s")
    run = functools.partial(
        pl.kernel,
        out_type=jax.ShapeDtypeStruct((32 * GPT, LANES), jnp.float32),
        mesh=mesh,
        scratch_types=[
            pltpu.VMEM((G_, LANES), jnp.float32),
            pltpu.VMEM((GC, S_ * L_), jnp.int32),
            pltpu.VMEM((GC, LANES), jnp.float32),
        ],
    )(_clause_body)
    out = run(xt, idx)

    return (out.reshape(NPART, NSLAB, GPT, LANES)
               .transpose(1, 3, 0, 2)
               .reshape(B_, G_))


# speedup vs baseline: 13.3055x; 13.3055x over previous
import functools

import jax
import jax.numpy as jnp
from jax import lax
from jax.experimental import pallas as pl
from jax.experimental.pallas import tpu as pltpu
from jax.experimental.pallas import tpu_sc as plsc

GAMMA_ = 0.01
B_, G_, S_, L_ = 64, 4096, 32, 4
LANES = 16
NSLAB = B_ // LANES
NPART = 32 // NSLAB
GPT = G_ // NPART
GC = 64
NCHUNK = GPT // GC

_LN2 = 0.6931471805599453
_INV_GAMMA = 1.0 / GAMMA_


def _sc_log(v):
    bits = lax.bitcast_convert_type(v, jnp.int32)
    e = ((bits >> 23) & 0xFF) - 127
    m = lax.bitcast_convert_type((bits & 0x007FFFFF) | 0x3F800000,
                                 jnp.float32)
    z = (m - 1.0) / (m + 1.0)
    z2 = z * z
    p = 1.0 / 9.0 + z2 * (1.0 / 11.0)
    p = 1.0 / 7.0 + z2 * p
    p = 1.0 / 5.0 + z2 * p
    p = 1.0 / 3.0 + z2 * p
    p = 1.0 + z2 * p
    return e.astype(jnp.float32) * _LN2 + 2.0 * z * p


def _clause_body(xt_hbm, idx_hbm, out_hbm, table_v, idx_v, out_v):
    wid = lax.axis_index("s") * 2 + lax.axis_index("c")
    bslab = lax.rem(wid, NSLAB)
    gpart = lax.div(wid, NSLAB)
    g0 = gpart * GPT

    pltpu.sync_copy(xt_hbm.at[bslab], table_v)

    @pl.loop(0, NCHUNK)
    def _chunk(c):
        gs = g0 + c * GC
        pltpu.sync_copy(idx_hbm.at[pl.ds(gs, GC)], idx_v)

        @pl.loop(0, GC)
        def _atom(gl):
            iv = [idx_v[gl, pl.ds(k * LANES, LANES)] for k in range(8)]
            conj = []
            mx = None
            for s in range(S_):
                p = 4 * s
                c0 = table_v[iv[p // 16][p % 16]]
                c1 = table_v[iv[(p + 1) // 16][(p + 1) % 16]]
                c2 = table_v[iv[(p + 2) // 16][(p + 2) % 16]]
                c3 = table_v[iv[(p + 3) // 16][(p + 3) % 16]]
                cv = (c0 * c1) * (c2 * c3)
                conj.append(cv)
                mx = cv if mx is None else jnp.maximum(mx, cv)
            acc = None
            for s in range(S_):
                ev = jnp.exp((conj[s] - mx) * _INV_GAMMA)
                acc = ev if acc is None else acc + ev
            out_v[gl, :] = mx + GAMMA_ * _sc_log(acc)

        pltpu.sync_copy(out_v, out_hbm.at[pl.ds(wid * GPT + c * GC, GC)])


@jax.jit
def kernel(x, I_i):
    xt = x.T.reshape(G_, NSLAB, LANES).transpose(1, 0, 2)
    idx = I_i.astype(jnp.int32).reshape(G_, S_ * L_)

    mesh = plsc.VectorSubcoreMesh(core_axis_name="c", subcore_axis_name="s")
    run = functools.partial(
        pl.kernel,
        out_type=jax.ShapeDtypeStruct((32 * GPT, LANES), jnp.float32),
        mesh=mesh,
        compiler_params=pltpu.CompilerParams(use_tc_tiling_on_sc=False),
        scratch_types=[
            pltpu.VMEM((G_, LANES), jnp.float32),
            pltpu.VMEM((GC, S_ * L_), jnp.int32),
            pltpu.VMEM((GC, LANES), jnp.float32),
        ],
    )(_clause_body)
    out = run(xt, idx)

    return (out.reshape(NPART, NSLAB, GPT, LANES)
               .transpose(1, 3, 0, 2)
               .reshape(B_, G_))
